# Initial kernel scaffold; baseline (speedup 1.0000x reference)
import jax
import jax.numpy as jnp
from jax.experimental import pallas as pl


def kernel(inputs, roi_label, roi_ious, lut, cq, reliability):
    raise NotImplementedError("write your pallas kernel here")


# speedup vs baseline: 3.2650x; 3.2650x over previous
import jax
import jax.numpy as jnp
from jax.experimental import pallas as pl
from jax.experimental.pallas import tpu as pltpu

NUM_FEATURES = 128
NUM_PIDS = 100000
NUM_CQ = 5000
OIM_SCALAR = 30.0
BATCH = 128
BLK = 5000
NUM_BLOCKS = NUM_PIDS // BLK
IGNORE_INDEX = 5554


def _oim_kernel(x_ref, lab_ref, lut_ref, cq_ref, rel_lut_ref, rel_cq_ref,
                out_ref, m_ref, s_ref, lsc_ref):
    i = pl.program_id(0)
    x = x_ref[...]
    labels = lab_ref[...]

    def scores(w, rel):
        lg = jax.lax.dot_general(
            x, w, (((1,), (1,)), ((), ())),
            preferred_element_type=jnp.float32,
            precision=jax.lax.Precision.HIGHEST)
        return lg * (rel * OIM_SCALAR)

    @pl.when(i == 0)
    def _init():
        cs = scores(cq_ref[...], rel_cq_ref[...])
        m0 = jnp.max(cs, axis=1, keepdims=True)
        m_ref[...] = m0
        s_ref[...] = jnp.sum(jnp.exp(cs - m0), axis=1, keepdims=True)
        lsc_ref[...] = jnp.zeros_like(lsc_ref)

    ls = scores(lut_ref[...], rel_lut_ref[0])
    bm = jnp.max(ls, axis=1, keepdims=True)
    m_old = m_ref[...]
    m_new = jnp.maximum(m_old, bm)
    s_ref[...] = (s_ref[...] * jnp.exp(m_old - m_new)
                  + jnp.sum(jnp.exp(ls - m_new), axis=1, keepdims=True))
    m_ref[...] = m_new

    col = jax.lax.broadcasted_iota(jnp.int32, (BATCH, BLK), 1) + i * BLK
    hit = col == labels
    lsc_ref[...] += jnp.sum(jnp.where(hit, ls, 0.0), axis=1, keepdims=True)

    @pl.when(i == NUM_BLOCKS - 1)
    def _finish():
        lse = m_ref[...] + jnp.log(s_ref[...])
        nll = lse - lsc_ref[...]
        valid = (labels != IGNORE_INDEX).astype(jnp.float32)
        loss = (jnp.sum(nll * valid, keepdims=True)
                / jnp.maximum(jnp.sum(valid, keepdims=True), 1.0))
        out_ref[...] = loss.reshape(1, 1)


def kernel(inputs, roi_label, roi_ious, lut, cq, reliability):
    del roi_ious
    labels = (roi_label.reshape(-1) - 1).astype(jnp.int32).reshape(BATCH, 1)
    rel_lut = reliability[:NUM_PIDS].reshape(NUM_BLOCKS, 1, BLK)
    rel_cq = reliability[NUM_PIDS:].reshape(1, NUM_CQ)

    out = pl.pallas_call(
        _oim_kernel,
        grid=(NUM_BLOCKS,),
        in_specs=[
            pl.BlockSpec((BATCH, NUM_FEATURES), lambda i: (0, 0)),
            pl.BlockSpec((BATCH, 1), lambda i: (0, 0)),
            pl.BlockSpec((BLK, NUM_FEATURES), lambda i: (i, 0)),
            pl.BlockSpec((NUM_CQ, NUM_FEATURES), lambda i: (0, 0)),
            pl.BlockSpec((1, 1, BLK), lambda i: (i, 0, 0)),
            pl.BlockSpec((1, NUM_CQ), lambda i: (0, 0)),
        ],
        out_specs=pl.BlockSpec((1, 1), lambda i: (0, 0)),
        out_shape=jax.ShapeDtypeStruct((1, 1), jnp.float32),
        scratch_shapes=[
            pltpu.VMEM((BATCH, 1), jnp.float32),
            pltpu.VMEM((BATCH, 1), jnp.float32),
            pltpu.VMEM((BATCH, 1), jnp.float32),
        ],
    )(inputs, labels, lut, cq, rel_lut, rel_cq)
    return out[0, 0]


# speedup vs baseline: 3.8620x; 1.1828x over previous
import jax
import jax.numpy as jnp
from jax.experimental import pallas as pl
from jax.experimental.pallas import tpu as pltpu

NUM_FEATURES = 128
NUM_PIDS = 100000
NUM_CQ = 5000
OIM_SCALAR = 30.0
BATCH = 128
BLK = 5000
NUM_BLOCKS = NUM_PIDS // BLK
IGNORE_INDEX = 5554


def _oim_kernel(x_ref, lab_ref, lut_ref, cq_ref, rel_lut_ref, rel_cq_ref,
                out_ref, s_ref, lsc_ref):
    i = pl.program_id(0)
    x = x_ref[...]
    labels = lab_ref[...]

    def scores(w, rel):
        lg = jax.lax.dot_general(
            x, w, (((1,), (1,)), ((), ())),
            preferred_element_type=jnp.float32,
            precision=jax.lax.Precision.HIGHEST)
        return lg * (rel * OIM_SCALAR)

    @pl.when(i == 0)
    def _init():
        cs = scores(cq_ref[...], rel_cq_ref[...])
        s_ref[...] = jnp.sum(jnp.exp(cs), axis=1, keepdims=True)
        lsc_ref[...] = jnp.zeros_like(lsc_ref)

    ls = scores(lut_ref[...], rel_lut_ref[0])
    s_ref[...] += jnp.sum(jnp.exp(ls), axis=1, keepdims=True)

    col = jax.lax.broadcasted_iota(jnp.int32, (BATCH, BLK), 1) + i * BLK
    hit = col == labels
    lsc_ref[...] += jnp.sum(jnp.where(hit, ls, 0.0), axis=1, keepdims=True)

    @pl.when(i == NUM_BLOCKS - 1)
    def _finish():
        lse = jnp.log(s_ref[...])
        nll = lse - lsc_ref[...]
        valid = (labels != IGNORE_INDEX).astype(jnp.float32)
        loss = (jnp.sum(nll * valid, keepdims=True)
                / jnp.maximum(jnp.sum(valid, keepdims=True), 1.0))
        out_ref[...] = loss.reshape(1, 1)


def kernel(inputs, roi_label, roi_ious, lut, cq, reliability):
    del roi_ious
    labels = (roi_label.reshape(-1) - 1).astype(jnp.int32).reshape(BATCH, 1)
    rel_lut = reliability[:NUM_PIDS].reshape(NUM_BLOCKS, 1, BLK)
    rel_cq = reliability[NUM_PIDS:].reshape(1, NUM_CQ)

    out = pl.pallas_call(
        _oim_kernel,
        grid=(NUM_BLOCKS,),
        in_specs=[
            pl.BlockSpec((BATCH, NUM_FEATURES), lambda i: (0, 0)),
            pl.BlockSpec((BATCH, 1), lambda i: (0, 0)),
            pl.BlockSpec((BLK, NUM_FEATURES), lambda i: (i, 0)),
            pl.BlockSpec((NUM_CQ, NUM_FEATURES), lambda i: (0, 0)),
            pl.BlockSpec((1, 1, BLK), lambda i: (i, 0, 0)),
            pl.BlockSpec((1, NUM_CQ), lambda i: (0, 0)),
        ],
        out_specs=pl.BlockSpec((1, 1), lambda i: (0, 0)),
        out_shape=jax.ShapeDtypeStruct((1, 1), jnp.float32),
        scratch_shapes=[
            pltpu.VMEM((BATCH, 1), jnp.float32),
            pltpu.VMEM((BATCH, 1), jnp.float32),
        ],
    )(inputs, labels, lut, cq, rel_lut, rel_cq)
    return out[0, 0]


# speedup vs baseline: 6.4998x; 1.6830x over previous
import jax
import jax.numpy as jnp
from jax.experimental import pallas as pl
from jax.experimental.pallas import tpu as pltpu

NUM_FEATURES = 128
NUM_PIDS = 100000
NUM_CQ = 5000
OIM_SCALAR = 30.0
BATCH = 128
BLK = 5000
NUM_BLOCKS = NUM_PIDS // BLK
IGNORE_INDEX = 5554


def _oim_kernel(x_ref, lab_ref, lut_ref, cq_ref, rel_lut_ref, rel_cq_ref,
                out_ref, s_ref, lsc_ref):
    i = pl.program_id(0)
    x = x_ref[...]
    labels = lab_ref[...]

    def scores(w, rel):
        lg = jax.lax.dot_general(
            x, w, (((1,), (1,)), ((), ())),
            preferred_element_type=jnp.float32,
            precision=jax.lax.Precision.DEFAULT)
        return lg * (rel * OIM_SCALAR)

    @pl.when(i == 0)
    def _init():
        cs = scores(cq_ref[...], rel_cq_ref[...])
        s_ref[...] = jnp.sum(jnp.exp(cs), axis=1, keepdims=True)
        lsc_ref[...] = jnp.zeros_like(lsc_ref)

    ls = scores(lut_ref[...], rel_lut_ref[0])
    s_ref[...] += jnp.sum(jnp.exp(ls), axis=1, keepdims=True)

    col = jax.lax.broadcasted_iota(jnp.int32, (BATCH, BLK), 1) + i * BLK
    hit = col == labels
    lsc_ref[...] += jnp.sum(jnp.where(hit, ls, 0.0), axis=1, keepdims=True)

    @pl.when(i == NUM_BLOCKS - 1)
    def _finish():
        lse = jnp.log(s_ref[...])
        nll = lse - lsc_ref[...]
        valid = (labels != IGNORE_INDEX).astype(jnp.float32)
        loss = (jnp.sum(nll * valid, keepdims=True)
                / jnp.maximum(jnp.sum(valid, keepdims=True), 1.0))
        out_ref[...] = loss.reshape(1, 1)


def kernel(inputs, roi_label, roi_ious, lut, cq, reliability):
    del roi_ious
    labels = (roi_label.reshape(-1) - 1).astype(jnp.int32).reshape(BATCH, 1)
    rel_lut = reliability[:NUM_PIDS].reshape(NUM_BLOCKS, 1, BLK)
    rel_cq = reliability[NUM_PIDS:].reshape(1, NUM_CQ)

    out = pl.pallas_call(
        _oim_kernel,
        grid=(NUM_BLOCKS,),
        in_specs=[
            pl.BlockSpec((BATCH, NUM_FEATURES), lambda i: (0, 0)),
            pl.BlockSpec((BATCH, 1), lambda i: (0, 0)),
            pl.BlockSpec((BLK, NUM_FEATURES), lambda i: (i, 0)),
            pl.BlockSpec((NUM_CQ, NUM_FEATURES), lambda i: (0, 0)),
            pl.BlockSpec((1, 1, BLK), lambda i: (i, 0, 0)),
            pl.BlockSpec((1, NUM_CQ), lambda i: (0, 0)),
        ],
        out_specs=pl.BlockSpec((1, 1), lambda i: (0, 0)),
        out_shape=jax.ShapeDtypeStruct((1, 1), jnp.float32),
        scratch_shapes=[
            pltpu.VMEM((BATCH, 1), jnp.float32),
            pltpu.VMEM((BATCH, 1), jnp.float32),
        ],
    )(inputs, labels, lut, cq, rel_lut, rel_cq)
    return out[0, 0]


# speedup vs baseline: 7.5415x; 1.1603x over previous
import jax
import jax.numpy as jnp
from jax.experimental import pallas as pl
from jax.experimental.pallas import tpu as pltpu

NUM_FEATURES = 128
NUM_PIDS = 100000
NUM_CQ = 5000
OIM_SCALAR = 30.0
BATCH = 128
BLK = 10000
NUM_BLOCKS = NUM_PIDS // BLK
IGNORE_INDEX = 5554


def _oim_kernel(x_ref, lab_ref, lut_ref, cq_ref, rel_lut_ref, rel_cq_ref,
                out_ref, s_ref, lsc_ref):
    i = pl.program_id(0)
    x = x_ref[...]
    labels = lab_ref[...]

    def scores(w, rel):
        lg = jax.lax.dot_general(
            x, w, (((1,), (1,)), ((), ())),
            preferred_element_type=jnp.float32,
            precision=jax.lax.Precision.DEFAULT)
        return lg * (rel * OIM_SCALAR)

    @pl.when(i == 0)
    def _init():
        cs = scores(cq_ref[...], rel_cq_ref[...])
        s_ref[...] = jnp.sum(jnp.exp(cs), axis=1, keepdims=True)
        lsc_ref[...] = jnp.zeros_like(lsc_ref)

    ls = scores(lut_ref[...], rel_lut_ref[0])
    s_ref[...] += jnp.sum(jnp.exp(ls), axis=1, keepdims=True)

    col = jax.lax.broadcasted_iota(jnp.int32, (BATCH, BLK), 1) + i * BLK
    hit = col == labels
    lsc_ref[...] += jnp.sum(jnp.where(hit, ls, 0.0), axis=1, keepdims=True)

    @pl.when(i == NUM_BLOCKS - 1)
    def _finish():
        lse = jnp.log(s_ref[...])
        nll = lse - lsc_ref[...]
        valid = (labels != IGNORE_INDEX).astype(jnp.float32)
        loss = (jnp.sum(nll * valid, keepdims=True)
                / jnp.maximum(jnp.sum(valid, keepdims=True), 1.0))
        out_ref[...] = loss.reshape(1, 1)


def kernel(inputs, roi_label, roi_ious, lut, cq, reliability):
    del roi_ious
    labels = (roi_label.reshape(-1) - 1).astype(jnp.int32).reshape(BATCH, 1)
    rel_lut = reliability[:NUM_PIDS].reshape(NUM_BLOCKS, 1, BLK)
    rel_cq = reliability[NUM_PIDS:].reshape(1, NUM_CQ)

    out = pl.pallas_call(
        _oim_kernel,
        grid=(NUM_BLOCKS,),
        in_specs=[
            pl.BlockSpec((BATCH, NUM_FEATURES), lambda i: (0, 0)),
            pl.BlockSpec((BATCH, 1), lambda i: (0, 0)),
            pl.BlockSpec((BLK, NUM_FEATURES), lambda i: (i, 0)),
            pl.BlockSpec((NUM_CQ, NUM_FEATURES), lambda i: (0, 0)),
            pl.BlockSpec((1, 1, BLK), lambda i: (i, 0, 0)),
            pl.BlockSpec((1, NUM_CQ), lambda i: (0, 0)),
        ],
        out_specs=pl.BlockSpec((1, 1), lambda i: (0, 0)),
        out_shape=jax.ShapeDtypeStruct((1, 1), jnp.float32),
        scratch_shapes=[
            pltpu.VMEM((BATCH, 1), jnp.float32),
            pltpu.VMEM((BATCH, 1), jnp.float32),
        ],
    )(inputs, labels, lut, cq, rel_lut, rel_cq)
    return out[0, 0]


# speedup vs baseline: 7.9215x; 1.0504x over previous
import jax
import jax.numpy as jnp
from jax.experimental import pallas as pl
from jax.experimental.pallas import tpu as pltpu

NUM_FEATURES = 128
NUM_PIDS = 100000
NUM_CQ = 5000
OIM_SCALAR = 30.0
BATCH = 128
BLK = 20000
NUM_BLOCKS = NUM_PIDS // BLK
IGNORE_INDEX = 5554


def _oim_kernel(x_ref, lab_ref, lut_ref, cq_ref, rel_lut_ref, rel_cq_ref,
                out_ref, s_ref, lsc_ref):
    i = pl.program_id(0)
    x = x_ref[...]
    labels = lab_ref[...]

    def scores(w, rel):
        lg = jax.lax.dot_general(
            x, w, (((1,), (1,)), ((), ())),
            preferred_element_type=jnp.float32,
            precision=jax.lax.Precision.DEFAULT)
        return lg * (rel * OIM_SCALAR)

    @pl.when(i == 0)
    def _init():
        cs = scores(cq_ref[...], rel_cq_ref[...])
        s_ref[...] = jnp.sum(jnp.exp(cs), axis=1, keepdims=True)
        lsc_ref[...] = jnp.zeros_like(lsc_ref)

    ls = scores(lut_ref[...], rel_lut_ref[0])
    s_ref[...] += jnp.sum(jnp.exp(ls), axis=1, keepdims=True)

    col = jax.lax.broadcasted_iota(jnp.int32, (BATCH, BLK), 1) + i * BLK
    hit = col == labels
    lsc_ref[...] += jnp.sum(jnp.where(hit, ls, 0.0), axis=1, keepdims=True)

    @pl.when(i == NUM_BLOCKS - 1)
    def _finish():
        lse = jnp.log(s_ref[...])
        nll = lse - lsc_ref[...]
        valid = (labels != IGNORE_INDEX).astype(jnp.float32)
        loss = (jnp.sum(nll * valid, keepdims=True)
                / jnp.maximum(jnp.sum(valid, keepdims=True), 1.0))
        out_ref[...] = loss.reshape(1, 1)


def kernel(inputs, roi_label, roi_ious, lut, cq, reliability):
    del roi_ious
    labels = (roi_label.reshape(-1) - 1).astype(jnp.int32).reshape(BATCH, 1)
    rel_lut = reliability[:NUM_PIDS].reshape(NUM_BLOCKS, 1, BLK)
    rel_cq = reliability[NUM_PIDS:].reshape(1, NUM_CQ)

    out = pl.pallas_call(
        _oim_kernel,
        grid=(NUM_BLOCKS,),
        in_specs=[
            pl.BlockSpec((BATCH, NUM_FEATURES), lambda i: (0, 0)),
            pl.BlockSpec((BATCH, 1), lambda i: (0, 0)),
            pl.BlockSpec((BLK, NUM_FEATURES), lambda i: (i, 0)),
            pl.BlockSpec((NUM_CQ, NUM_FEATURES), lambda i: (0, 0)),
            pl.BlockSpec((1, 1, BLK), lambda i: (i, 0, 0)),
            pl.BlockSpec((1, NUM_CQ), lambda i: (0, 0)),
        ],
        out_specs=pl.BlockSpec((1, 1), lambda i: (0, 0)),
        out_shape=jax.ShapeDtypeStruct((1, 1), jnp.float32),
        scratch_shapes=[
            pltpu.VMEM((BATCH, 1), jnp.float32),
            pltpu.VMEM((BATCH, 1), jnp.float32),
        ],
    )(inputs, labels, lut, cq, rel_lut, rel_cq)
    return out[0, 0]
